# baseline (speedup 1.0000x reference)
import functools

import jax
import jax.numpy as jnp
from jax import lax
from jax.experimental import pallas as pl
from jax.experimental.pallas import tpu as pltpu
from jax.experimental.pallas import tpu_sc as plsc

B = 16384
D = 64
PR = 2 * D
LANES = 16
NC, NS = 2, 16
NW = NC * NS
ROWS = B // NW
CHUNK = 64
NCHUNK = ROWS // CHUNK
MARGIN = 1.0
K = D // LANES


def _rsqrt(x):
    i = plsc.bitcast(x, jnp.int32)
    i = jnp.int32(0x5F3759DF) - lax.shift_right_logical(i, 1)
    y = plsc.bitcast(i, jnp.float32)
    for _ in range(3):
        y = y * (1.5 - 0.5 * x * y * y)
    return y


def _transfer_row(e_buf, t_buf, rtk, pe, r):
    ek = [e_buf[r, pl.ds(pe + 16 * k, 16)] for k in range(K)]
    tk = [t_buf[r, pl.ds(pe + 16 * k, 16)] for k in range(K)]
    d = ek[0] * tk[0]
    for k in range(1, K):
        d = d + ek[k] * tk[k]
    dsum = jnp.full((LANES,), jnp.sum(d), jnp.float32)
    vk = [ek[k] + dsum * rtk[k] for k in range(K)]
    s2 = vk[0] * vk[0]
    for k in range(1, K):
        s2 = s2 + vk[k] * vk[k]
    s2s = jnp.full((LANES,), jnp.sum(s2), jnp.float32)
    y = _rsqrt(s2s)
    norm = s2s * y
    inv = 1.0 / jnp.maximum(norm, 1e-12)
    return [vk[k] * inv for k in range(K)]


def _side_score(heb, htb, teb, ttb, reb, rtb, ph_, pt_, pr_, r):
    rtk = [rtb[r, pl.ds(pr_ + 16 * k, 16)] for k in range(K)]
    hk = _transfer_row(heb, htb, rtk, ph_, r)
    tk = _transfer_row(teb, ttb, rtk, pt_, r)
    acc = None
    for k in range(K):
        rek = reb[r, pl.ds(pr_ + 16 * k, 16)]
        term = jnp.abs(hk[k] + rek - tk[k])
        acc = term if acc is None else acc + term
    return jnp.sum(acc)


def _disc_body(iph, ipt, ipr, inh, intt, inr,
               aph, apt, apr, anh, antt, anr, takef,
               ent_emb, rel_emb, ent_tr, rel_tr,
               nscore_out, partial_out,
               iv0, iv1, iv2, iv3, iv4, iv5,
               pv0, pv1, pv2, pv3, pv4, pv5,
               phe, pht, pte, ptt, pre, prt,
               nhe, nht, nte, ntt, nre, nrt,
               take_v, ns_buf, loss_buf, isem, gsem):
    idx_v = [iv0, iv1, iv2, iv3, iv4, iv5]
    par_v = [pv0, pv1, pv2, pv3, pv4, pv5]
    wid = lax.axis_index("s") * NC + lax.axis_index("c")
    base = wid * ROWS
    lane = lax.iota(jnp.int32, LANES)

    pltpu.sync_copy(takef.at[pl.ds(base, ROWS)], take_v)

    lossv = jnp.zeros((LANES,), jnp.float32)
    for c in range(NCHUNK):
        off = base + c * CHUNK
        idx_cps = [
            pltpu.async_copy(src.at[pl.ds(off, CHUNK)], dst, isem)
            for src, dst in ((iph, idx_v[0]), (ipt, idx_v[1]),
                             (ipr, idx_v[2]), (inh, idx_v[3]),
                             (intt, idx_v[4]), (inr, idx_v[5]),
                             (aph, par_v[0]), (apt, par_v[1]),
                             (apr, par_v[2]), (anh, par_v[3]),
                             (antt, par_v[4]), (anr, par_v[5]))
        ]
        for cp in idx_cps:
            cp.wait()
        gathers = [
            pltpu.async_copy(tab.at[idx_v[j]], dst, gsem)
            for tab, j, dst in (
                (ent_emb, 0, phe), (ent_tr, 0, pht),
                (ent_emb, 1, pte), (ent_tr, 1, ptt),
                (rel_emb, 2, pre), (rel_tr, 2, prt),
                (ent_emb, 3, nhe), (ent_tr, 3, nht),
                (ent_emb, 4, nte), (ent_tr, 4, ntt),
                (rel_emb, 5, nre), (rel_tr, 5, nrt),
            )
        ]
        for cp in gathers:
            cp.wait()

        def group_body(g, lossv_c):
            pv = [par_v[j][pl.ds(g * LANES, LANES)] for j in range(6)]
            zi = jnp.zeros((LANES,), jnp.int32)

            def row_body(i, carry):
                nsv, psv = carry
                r = g * LANES + i
                onehot = lane == i
                pars = [jnp.sum(jnp.where(onehot, pv[j], zi)) for j in range(6)]
                p_s = _side_score(phe, pht, pte, ptt, pre, prt,
                                  pars[0], pars[1], pars[2], r)
                n_s = _side_score(nhe, nht, nte, ntt, nre, nrt,
                                  pars[3], pars[4], pars[5], r)
                nsv = jnp.where(onehot,
                                jnp.full((LANES,), -n_s, jnp.float32), nsv)
                psv = jnp.where(onehot,
                                jnp.full((LANES,), p_s, jnp.float32), psv)
                return nsv, psv

            z = jnp.zeros((LANES,), jnp.float32)
            nsv, psv = lax.fori_loop(0, LANES, row_body, (z, z))
            tkv = take_v[pl.ds(c * CHUNK + g * LANES, LANES)]
            lossv_c = lossv_c + jnp.maximum(0.0, psv + nsv + MARGIN) * tkv
            ns_buf[pl.ds(c * CHUNK + g * LANES, LANES)] = nsv
            return lossv_c

        lossv = lax.fori_loop(0, CHUNK // LANES, group_body, lossv)

    pltpu.sync_copy(ns_buf, nscore_out.at[pl.ds(base, ROWS)])
    loss_buf[...] = lossv
    pltpu.sync_copy(loss_buf, partial_out.at[wid])


_disc = functools.partial(
    pl.kernel,
    mesh=plsc.VectorSubcoreMesh(core_axis_name="c", subcore_axis_name="s"),
    compiler_params=pltpu.CompilerParams(needs_layout_passes=False),
    out_type=[
        jax.ShapeDtypeStruct((B,), jnp.float32),
        jax.ShapeDtypeStruct((NW, LANES), jnp.float32),
    ],
    scratch_types=(
        [pltpu.VMEM((CHUNK,), jnp.int32) for _ in range(12)]
        + [pltpu.VMEM((CHUNK, PR), jnp.float32) for _ in range(12)]
        + [pltpu.VMEM((ROWS,), jnp.float32),
           pltpu.VMEM((ROWS,), jnp.float32),
           pltpu.VMEM((LANES,), jnp.float32),
           pltpu.SemaphoreType.DMA,
           pltpu.SemaphoreType.DMA]
    ),
)(_disc_body)


def _sum_body(p_ref, o_ref):
    o_ref[0, 0] = jnp.sum(p_ref[...])


_sum_partials = pl.pallas_call(
    _sum_body,
    out_shape=jax.ShapeDtypeStruct((1, 1), jnp.float32),
    out_specs=pl.BlockSpec(memory_space=pltpu.SMEM),
)


def kernel(pos_h, pos_r, pos_t, neg_h, neg_r, neg_t, take,
           ent_emb_w, rel_emb_w, ent_transfer_w, rel_transfer_w):
    srcs = [pos_h, pos_t, pos_r, neg_h, neg_t, neg_r]
    phys = [lax.shift_right_logical(s.astype(jnp.int32), 1) for s in srcs]
    pars = [(s.astype(jnp.int32) & 1) * D for s in srcs]
    takef = take.astype(jnp.float32)
    ent2 = ent_emb_w.reshape(-1, PR)
    etr2 = ent_transfer_w.reshape(-1, PR)
    rel2 = rel_emb_w.reshape(-1, PR)
    rtr2 = rel_transfer_w.reshape(-1, PR)
    nscore, partials = _disc(*phys, *pars, takef,
                             ent2, rel2, etr2, rtr2)
    loss = _sum_partials(partials)[0, 0]
    return (loss, nscore)


# speedup vs baseline: 1.1895x; 1.1895x over previous
import functools

import jax
import jax.numpy as jnp
from jax import lax
from jax.experimental import pallas as pl
from jax.experimental.pallas import tpu as pltpu
from jax.experimental.pallas import tpu_sc as plsc

B = 16384
D = 64
PR = 2 * D
LANES = 16
NC, NS = 2, 16
NW = NC * NS
ROWS = B // NW
CHUNK = 64
NCHUNK = ROWS // CHUNK
MARGIN = 1.0
K = D // LANES


def _rsqrt(x):
    i = plsc.bitcast(x, jnp.int32)
    i = jnp.int32(0x5F3759DF) - lax.shift_right_logical(i, 1)
    y = plsc.bitcast(i, jnp.float32)
    for _ in range(3):
        y = y * (1.5 - 0.5 * x * y * y)
    return y


def _transfer_row(e_buf, t_buf, rtk, pe, r):
    ek = [e_buf[r, pl.ds(pe + 16 * k, 16)] for k in range(K)]
    tk = [t_buf[r, pl.ds(pe + 16 * k, 16)] for k in range(K)]
    d = ek[0] * tk[0]
    for k in range(1, K):
        d = d + ek[k] * tk[k]
    dsum = jnp.full((LANES,), jnp.sum(d), jnp.float32)
    vk = [ek[k] + dsum * rtk[k] for k in range(K)]
    s2 = vk[0] * vk[0]
    for k in range(1, K):
        s2 = s2 + vk[k] * vk[k]
    s2s = jnp.full((LANES,), jnp.sum(s2), jnp.float32)
    y = _rsqrt(s2s)
    norm = s2s * y
    inv = 1.0 / jnp.maximum(norm, 1e-12)
    return [vk[k] * inv for k in range(K)]


def _side_score(heb, htb, teb, ttb, reb, rtb, ph_, pt_, pr_, r):
    rtk = [rtb[r, pl.ds(pr_ + 16 * k, 16)] for k in range(K)]
    hk = _transfer_row(heb, htb, rtk, ph_, r)
    tk = _transfer_row(teb, ttb, rtk, pt_, r)
    acc = None
    for k in range(K):
        rek = reb[r, pl.ds(pr_ + 16 * k, 16)]
        term = jnp.abs(hk[k] + rek - tk[k])
        acc = term if acc is None else acc + term
    return jnp.sum(acc)


def _disc_body(iph, ipt, ipr, inh, intt, inr,
               aph, apt, apr, anh, antt, anr, takef,
               ent_emb, rel_emb, ent_tr, rel_tr,
               nscore_out, partial_out,
               iv0, iv1, iv2, iv3, iv4, iv5,
               pv0, pv1, pv2, pv3, pv4, pv5,
               phe, pht, pte, ptt, pre, prt,
               nhe, nht, nte, ntt, nre, nrt,
               take_v, ns_buf, loss_buf, isem, gsem):
    idx_v = [iv0, iv1, iv2, iv3, iv4, iv5]
    par_v = [pv0, pv1, pv2, pv3, pv4, pv5]
    wid = lax.axis_index("s") * NC + lax.axis_index("c")
    base = wid * ROWS
    lane = lax.iota(jnp.int32, LANES)

    pltpu.sync_copy(takef.at[pl.ds(base, ROWS)], take_v)

    lossv = jnp.zeros((LANES,), jnp.float32)
    for c in range(NCHUNK):
        off = base + c * CHUNK
        idx_cps = [
            pltpu.async_copy(src.at[pl.ds(off, CHUNK)], dst, isem)
            for src, dst in ((iph, idx_v[0]), (ipt, idx_v[1]),
                             (ipr, idx_v[2]), (inh, idx_v[3]),
                             (intt, idx_v[4]), (inr, idx_v[5]),
                             (aph, par_v[0]), (apt, par_v[1]),
                             (apr, par_v[2]), (anh, par_v[3]),
                             (antt, par_v[4]), (anr, par_v[5]))
        ]
        for cp in idx_cps:
            cp.wait()
        gathers = [
            pltpu.async_copy(tab.at[idx_v[j]], dst, gsem)
            for tab, j, dst in (
                (ent_emb, 0, phe), (ent_tr, 0, pht),
                (ent_emb, 1, pte), (ent_tr, 1, ptt),
                (rel_emb, 2, pre), (rel_tr, 2, prt),
                (ent_emb, 3, nhe), (ent_tr, 3, nht),
                (ent_emb, 4, nte), (ent_tr, 4, ntt),
                (rel_emb, 5, nre), (rel_tr, 5, nrt),
            )
        ]
        for cp in gathers:
            cp.wait()

        def group_body(g, lossv_c):
            pv = [par_v[j][pl.ds(g * LANES, LANES)] for j in range(6)]
            zi = jnp.zeros((LANES,), jnp.int32)

            def row_body(i, carry):
                nsv, psv = carry
                r = g * LANES + i
                onehot = lane == i
                pars = [jnp.sum(jnp.where(onehot, pv[j], zi)) for j in range(6)]
                p_s = _side_score(phe, pht, pte, ptt, pre, prt,
                                  pars[0], pars[1], pars[2], r)
                n_s = _side_score(nhe, nht, nte, ntt, nre, nrt,
                                  pars[3], pars[4], pars[5], r)
                nsv = jnp.where(onehot,
                                jnp.full((LANES,), -n_s, jnp.float32), nsv)
                psv = jnp.where(onehot,
                                jnp.full((LANES,), p_s, jnp.float32), psv)
                return nsv, psv

            z = jnp.zeros((LANES,), jnp.float32)
            nsv, psv = lax.fori_loop(0, LANES, row_body, (z, z))
            tkv = take_v[pl.ds(c * CHUNK + g * LANES, LANES)]
            lossv_c = lossv_c + jnp.maximum(0.0, psv + nsv + MARGIN) * tkv
            ns_buf[pl.ds(c * CHUNK + g * LANES, LANES)] = nsv
            return lossv_c

        lossv = lax.fori_loop(0, CHUNK // LANES, group_body, lossv)

    pltpu.sync_copy(ns_buf, nscore_out.at[pl.ds(base, ROWS)])
    loss_buf[...] = lossv
    pltpu.sync_copy(loss_buf, partial_out.at[wid])


_disc = functools.partial(
    pl.kernel,
    mesh=plsc.VectorSubcoreMesh(core_axis_name="c", subcore_axis_name="s"),
    compiler_params=pltpu.CompilerParams(needs_layout_passes=False),
    out_type=[
        jax.ShapeDtypeStruct((B,), jnp.float32),
        jax.ShapeDtypeStruct((NW, LANES), jnp.float32),
    ],
    scratch_types=(
        [pltpu.VMEM((CHUNK,), jnp.int32) for _ in range(12)]
        + [pltpu.VMEM((CHUNK, PR), jnp.float32) for _ in range(12)]
        + [pltpu.VMEM((ROWS,), jnp.float32),
           pltpu.VMEM((ROWS,), jnp.float32),
           pltpu.VMEM((LANES,), jnp.float32),
           pltpu.SemaphoreType.DMA,
           pltpu.SemaphoreType.DMA]
    ),
)(_disc_body)


def _repack_body(x_ref, o_ref, *, half):
    xt = x_ref[...].T
    o_ref[:, 0:64] = xt[0:half]
    o_ref[:, 64:128] = xt[half:2 * half]


def _make_repack(n_rows, blk):
    half = blk // 2
    grid = -(-n_rows // blk)
    return pl.pallas_call(
        functools.partial(_repack_body, half=half),
        grid=(grid,),
        in_specs=[pl.BlockSpec((D, blk), lambda i: (0, i))],
        out_specs=pl.BlockSpec((half, PR), lambda i: (i, 0)),
        out_shape=jax.ShapeDtypeStruct((grid * half, PR), jnp.float32),
    )


_ENT_BLK = 2048
_REL_BLK = 1000
_repack_ent = _make_repack(1000000, _ENT_BLK)
_repack_rel = _make_repack(1000, _REL_BLK)


def _sum_body(p_ref, o_ref):
    o_ref[0, 0] = jnp.sum(p_ref[...])


_sum_partials = pl.pallas_call(
    _sum_body,
    out_shape=jax.ShapeDtypeStruct((1, 1), jnp.float32),
    out_specs=pl.BlockSpec(memory_space=pltpu.SMEM),
)


def kernel(pos_h, pos_r, pos_t, neg_h, neg_r, neg_t, take,
           ent_emb_w, rel_emb_w, ent_transfer_w, rel_transfer_w):
    def map_idx(s, blk):
        s = s.astype(jnp.int32)
        half = blk // 2
        u = s % blk
        return (s // blk) * half + u % half, (u >= half).astype(jnp.int32) * D

    ent_srcs = [pos_h, pos_t, neg_h, neg_t]
    rel_srcs = [pos_r, neg_r]
    mph, aph = map_idx(pos_h, _ENT_BLK)
    mpt, apt = map_idx(pos_t, _ENT_BLK)
    mnh, anh = map_idx(neg_h, _ENT_BLK)
    mnt, ant = map_idx(neg_t, _ENT_BLK)
    mpr, apr = map_idx(pos_r, _REL_BLK)
    mnr, anr = map_idx(neg_r, _REL_BLK)
    phys = [mph, mpt, mpr, mnh, mnt, mnr]
    pars = [aph, apt, apr, anh, ant, anr]
    takef = take.astype(jnp.float32)
    ent2 = _repack_ent(ent_emb_w.T)
    etr2 = _repack_ent(ent_transfer_w.T)
    rel2 = _repack_rel(rel_emb_w.T)
    rtr2 = _repack_rel(rel_transfer_w.T)
    nscore, partials = _disc(*phys, *pars, takef,
                             ent2, rel2, etr2, rtr2)
    loss = _sum_partials(partials)[0, 0]
    return (loss, nscore)


# speedup vs baseline: 1.9048x; 1.6013x over previous
import functools

import jax
import jax.numpy as jnp
from jax import lax
from jax.experimental import pallas as pl
from jax.experimental.pallas import tpu as pltpu
from jax.experimental.pallas import tpu_sc as plsc

B = 16384
D = 64
PR = 2 * D
LANES = 16
NC, NS = 2, 16
NW = NC * NS
ROWS = B // NW
CHUNK = 64
NCHUNK = ROWS // CHUNK
MARGIN = 1.0
K = D // LANES


def _rsqrt(x):
    i = plsc.bitcast(x, jnp.int32)
    i = jnp.int32(0x5F3759DF) - lax.shift_right_logical(i, 1)
    y = plsc.bitcast(i, jnp.float32)
    for _ in range(3):
        y = y * (1.5 - 0.5 * x * y * y)
    return y


def _transfer_row(e_buf, t_buf, rtk, pe, r):
    ek = [e_buf[r, pl.ds(pe + 16 * k, 16)] for k in range(K)]
    tk = [t_buf[r, pl.ds(pe + 16 * k, 16)] for k in range(K)]
    d = ek[0] * tk[0]
    for k in range(1, K):
        d = d + ek[k] * tk[k]
    dsum = jnp.full((LANES,), jnp.sum(d), jnp.float32)
    vk = [ek[k] + dsum * rtk[k] for k in range(K)]
    s2 = vk[0] * vk[0]
    for k in range(1, K):
        s2 = s2 + vk[k] * vk[k]
    s2s = jnp.full((LANES,), jnp.sum(s2), jnp.float32)
    y = _rsqrt(s2s)
    norm = s2s * y
    inv = 1.0 / jnp.maximum(norm, 1e-12)
    return [vk[k] * inv for k in range(K)]


def _side_score(heb, htb, teb, ttb, reb, rtb, ph_, pt_, pr_, r):
    rtk = [rtb[r, pl.ds(pr_ + 16 * k, 16)] for k in range(K)]
    hk = _transfer_row(heb, htb, rtk, ph_, r)
    tk = _transfer_row(teb, ttb, rtk, pt_, r)
    acc = None
    for k in range(K):
        rek = reb[r, pl.ds(pr_ + 16 * k, 16)]
        term = jnp.abs(hk[k] + rek - tk[k])
        acc = term if acc is None else acc + term
    return jnp.sum(acc)


def _disc_body(iph, ipt, ipr, inh, intt, inr,
               aph, apt, apr, anh, antt, anr, takef,
               ent_emb, rel_emb, ent_tr, rel_tr,
               nscore_out, partial_out,
               iv0, iv1, iv2, iv3, iv4, iv5,
               pv0, pv1, pv2, pv3, pv4, pv5,
               phe, pht, pte, ptt, pre, prt,
               nhe, nht, nte, ntt, nre, nrt,
               take_v, ns_buf, loss_buf, isem, gsem):
    idx_v = [iv0, iv1, iv2, iv3, iv4, iv5]
    par_v = [pv0, pv1, pv2, pv3, pv4, pv5]
    wid = lax.axis_index("s") * NC + lax.axis_index("c")
    base = wid * ROWS
    lane = lax.iota(jnp.int32, LANES)

    pltpu.sync_copy(takef.at[pl.ds(base, ROWS)], take_v)

    lossv = jnp.zeros((LANES,), jnp.float32)
    for c in range(NCHUNK):
        off = base + c * CHUNK
        idx_cps = [
            pltpu.async_copy(src.at[pl.ds(off, CHUNK)], dst, isem)
            for src, dst in ((iph, idx_v[0]), (ipt, idx_v[1]),
                             (ipr, idx_v[2]), (inh, idx_v[3]),
                             (intt, idx_v[4]), (inr, idx_v[5]),
                             (aph, par_v[0]), (apt, par_v[1]),
                             (apr, par_v[2]), (anh, par_v[3]),
                             (antt, par_v[4]), (anr, par_v[5]))
        ]
        for cp in idx_cps:
            cp.wait()
        gathers = [
            pltpu.async_copy(tab.at[idx_v[j]], dst, gsem)
            for tab, j, dst in (
                (ent_emb, 0, phe), (ent_tr, 0, pht),
                (ent_emb, 1, pte), (ent_tr, 1, ptt),
                (rel_emb, 2, pre), (rel_tr, 2, prt),
                (ent_emb, 3, nhe), (ent_tr, 3, nht),
                (ent_emb, 4, nte), (ent_tr, 4, ntt),
                (rel_emb, 5, nre), (rel_tr, 5, nrt),
            )
        ]
        for cp in gathers:
            cp.wait()

        def group_body(g, lossv_c):
            pv = [par_v[j][pl.ds(g * LANES, LANES)] for j in range(6)]
            zi = jnp.zeros((LANES,), jnp.int32)

            def row_body(i, carry):
                nsv, psv = carry
                r = g * LANES + i
                onehot = lane == i
                pars = [jnp.sum(jnp.where(onehot, pv[j], zi)) for j in range(6)]
                p_s = _side_score(phe, pht, pte, ptt, pre, prt,
                                  pars[0], pars[1], pars[2], r)
                n_s = _side_score(nhe, nht, nte, ntt, nre, nrt,
                                  pars[3], pars[4], pars[5], r)
                nsv = jnp.where(onehot,
                                jnp.full((LANES,), -n_s, jnp.float32), nsv)
                psv = jnp.where(onehot,
                                jnp.full((LANES,), p_s, jnp.float32), psv)
                return nsv, psv

            z = jnp.zeros((LANES,), jnp.float32)
            nsv, psv = lax.fori_loop(0, LANES, row_body, (z, z))
            tkv = take_v[pl.ds(c * CHUNK + g * LANES, LANES)]
            lossv_c = lossv_c + jnp.maximum(0.0, psv + nsv + MARGIN) * tkv
            ns_buf[pl.ds(c * CHUNK + g * LANES, LANES)] = nsv
            return lossv_c

        lossv = lax.fori_loop(0, CHUNK // LANES, group_body, lossv)

    pltpu.sync_copy(ns_buf, nscore_out.at[pl.ds(base, ROWS)])
    loss_buf[...] = lossv
    pltpu.sync_copy(loss_buf, partial_out.at[wid])


_disc = functools.partial(
    pl.kernel,
    mesh=plsc.VectorSubcoreMesh(core_axis_name="c", subcore_axis_name="s"),
    compiler_params=pltpu.CompilerParams(needs_layout_passes=False),
    out_type=[
        jax.ShapeDtypeStruct((B,), jnp.float32),
        jax.ShapeDtypeStruct((NW, LANES), jnp.float32),
    ],
    scratch_types=(
        [pltpu.VMEM((CHUNK,), jnp.int32) for _ in range(12)]
        + [pltpu.VMEM((CHUNK, PR), jnp.float32) for _ in range(12)]
        + [pltpu.VMEM((ROWS,), jnp.float32),
           pltpu.VMEM((ROWS,), jnp.float32),
           pltpu.VMEM((LANES,), jnp.float32),
           pltpu.SemaphoreType.DMA,
           pltpu.SemaphoreType.DMA]
    ),
)(_disc_body)


def _repack_body(x_ref, o_ref, *, half):
    xt = x_ref[...].T
    o_ref[:, 0:64] = xt[0:half]
    o_ref[:, 64:128] = xt[half:2 * half]


def _make_repack(n_rows, blk):
    half = blk // 2
    grid = -(-n_rows // blk)
    return pl.pallas_call(
        functools.partial(_repack_body, half=half),
        grid=(grid,),
        in_specs=[pl.BlockSpec((D, blk), lambda i: (0, i))],
        out_specs=pl.BlockSpec((half, PR), lambda i: (i, 0)),
        out_shape=jax.ShapeDtypeStruct((grid * half, PR), jnp.float32),
    )


_ENT_BLK = 8192
_REL_BLK = 1000
_repack_ent = _make_repack(1000000, _ENT_BLK)
_repack_rel = _make_repack(1000, _REL_BLK)


def _sum_body(p_ref, o_ref):
    o_ref[0, 0] = jnp.sum(p_ref[...])


_sum_partials = pl.pallas_call(
    _sum_body,
    out_shape=jax.ShapeDtypeStruct((1, 1), jnp.float32),
    out_specs=pl.BlockSpec(memory_space=pltpu.SMEM),
)


def kernel(pos_h, pos_r, pos_t, neg_h, neg_r, neg_t, take,
           ent_emb_w, rel_emb_w, ent_transfer_w, rel_transfer_w):
    def map_idx(s, blk):
        s = s.astype(jnp.int32)
        half = blk // 2
        u = s % blk
        return (s // blk) * half + u % half, (u >= half).astype(jnp.int32) * D

    ent_srcs = [pos_h, pos_t, neg_h, neg_t]
    rel_srcs = [pos_r, neg_r]
    mph, aph = map_idx(pos_h, _ENT_BLK)
    mpt, apt = map_idx(pos_t, _ENT_BLK)
    mnh, anh = map_idx(neg_h, _ENT_BLK)
    mnt, ant = map_idx(neg_t, _ENT_BLK)
    mpr, apr = map_idx(pos_r, _REL_BLK)
    mnr, anr = map_idx(neg_r, _REL_BLK)
    phys = [mph, mpt, mpr, mnh, mnt, mnr]
    pars = [aph, apt, apr, anh, ant, anr]
    takef = take.astype(jnp.float32)
    ent2 = _repack_ent(ent_emb_w.T)
    etr2 = _repack_ent(ent_transfer_w.T)
    rel2 = _repack_rel(rel_emb_w.T)
    rtr2 = _repack_rel(rel_transfer_w.T)
    nscore, partials = _disc(*phys, *pars, takef,
                             ent2, rel2, etr2, rtr2)
    loss = _sum_partials(partials)[0, 0]
    return (loss, nscore)


# speedup vs baseline: 2.2493x; 1.1809x over previous
import functools

import jax
import jax.numpy as jnp
from jax import lax
from jax.experimental import pallas as pl
from jax.experimental.pallas import tpu as pltpu
from jax.experimental.pallas import tpu_sc as plsc

B = 16384
D = 64
PR = 2 * D
LANES = 16
NC, NS = 2, 16
NW = NC * NS
ROWS = B // NW
CHUNK = 64
NCHUNK = ROWS // CHUNK
MARGIN = 1.0
K = D // LANES


def _rsqrt(x):
    i = plsc.bitcast(x, jnp.int32)
    i = jnp.int32(0x5F3759DF) - lax.shift_right_logical(i, 1)
    y = plsc.bitcast(i, jnp.float32)
    for _ in range(3):
        y = y * (1.5 - 0.5 * x * y * y)
    return y


def _transfer_row(e_buf, t_buf, rtk, pe, r):
    ek = [e_buf[r, pl.ds(pe + 16 * k, 16)] for k in range(K)]
    tk = [t_buf[r, pl.ds(pe + 16 * k, 16)] for k in range(K)]
    d = ek[0] * tk[0]
    for k in range(1, K):
        d = d + ek[k] * tk[k]
    dsum = jnp.full((LANES,), jnp.sum(d), jnp.float32)
    vk = [ek[k] + dsum * rtk[k] for k in range(K)]
    s2 = vk[0] * vk[0]
    for k in range(1, K):
        s2 = s2 + vk[k] * vk[k]
    s2s = jnp.full((LANES,), jnp.sum(s2), jnp.float32)
    y = _rsqrt(s2s)
    norm = s2s * y
    inv = 1.0 / jnp.maximum(norm, 1e-12)
    return [vk[k] * inv for k in range(K)]


def _side_score(heb, htb, teb, ttb, reb, rtb, ph_, pt_, pr_, r):
    rtk = [rtb[r, pl.ds(pr_ + 16 * k, 16)] for k in range(K)]
    hk = _transfer_row(heb, htb, rtk, ph_, r)
    tk = _transfer_row(teb, ttb, rtk, pt_, r)
    acc = None
    for k in range(K):
        rek = reb[r, pl.ds(pr_ + 16 * k, 16)]
        term = jnp.abs(hk[k] + rek - tk[k])
        acc = term if acc is None else acc + term
    return jnp.sum(acc)


def _disc_body(iph, ipt, ipr, inh, intt, inr,
               aph, apt, apr, anh, antt, anr, takef,
               ent_emb, rel_emb, ent_tr, rel_tr,
               nscore_out, partial_out,
               iv0, iv1, iv2, iv3, iv4, iv5,
               pv0, pv1, pv2, pv3, pv4, pv5,
               phe, pht, pte, ptt, pre, prt,
               nhe, nht, nte, ntt, nre, nrt,
               take_v, ns_buf, loss_buf, isem, gsem):
    idx_v = [iv0, iv1, iv2, iv3, iv4, iv5]
    par_v = [pv0, pv1, pv2, pv3, pv4, pv5]
    wid = lax.axis_index("s") * NC + lax.axis_index("c")
    base = wid * ROWS
    lane = lax.iota(jnp.int32, LANES)

    pltpu.sync_copy(takef.at[pl.ds(base, ROWS)], take_v)

    lossv = jnp.zeros((LANES,), jnp.float32)
    for c in range(NCHUNK):
        off = base + c * CHUNK
        idx_cps = [
            pltpu.async_copy(src.at[pl.ds(off, CHUNK)], dst, isem)
            for src, dst in ((iph, idx_v[0]), (ipt, idx_v[1]),
                             (ipr, idx_v[2]), (inh, idx_v[3]),
                             (intt, idx_v[4]), (inr, idx_v[5]),
                             (aph, par_v[0]), (apt, par_v[1]),
                             (apr, par_v[2]), (anh, par_v[3]),
                             (antt, par_v[4]), (anr, par_v[5]))
        ]
        for cp in idx_cps:
            cp.wait()
        gathers = [
            pltpu.async_copy(tab.at[idx_v[j]], dst, gsem)
            for tab, j, dst in (
                (ent_emb, 0, phe), (ent_tr, 0, pht),
                (ent_emb, 1, pte), (ent_tr, 1, ptt),
                (rel_emb, 2, pre), (rel_tr, 2, prt),
                (ent_emb, 3, nhe), (ent_tr, 3, nht),
                (ent_emb, 4, nte), (ent_tr, 4, ntt),
                (rel_emb, 5, nre), (rel_tr, 5, nrt),
            )
        ]
        for cp in gathers:
            cp.wait()

        def group_body(g, lossv_c):
            pv = [par_v[j][pl.ds(g * LANES, LANES)] for j in range(6)]
            zi = jnp.zeros((LANES,), jnp.int32)

            def row_body(i, carry):
                nsv, psv = carry
                r = g * LANES + i
                onehot = lane == i
                pars = [jnp.sum(jnp.where(onehot, pv[j], zi)) for j in range(6)]
                p_s = _side_score(phe, pht, pte, ptt, pre, prt,
                                  pars[0], pars[1], pars[2], r)
                n_s = _side_score(nhe, nht, nte, ntt, nre, nrt,
                                  pars[3], pars[4], pars[5], r)
                nsv = jnp.where(onehot,
                                jnp.full((LANES,), -n_s, jnp.float32), nsv)
                psv = jnp.where(onehot,
                                jnp.full((LANES,), p_s, jnp.float32), psv)
                return nsv, psv

            z = jnp.zeros((LANES,), jnp.float32)
            nsv, psv = lax.fori_loop(0, LANES, row_body, (z, z))
            tkv = take_v[pl.ds(c * CHUNK + g * LANES, LANES)]
            lossv_c = lossv_c + jnp.maximum(0.0, psv + nsv + MARGIN) * tkv
            ns_buf[pl.ds(c * CHUNK + g * LANES, LANES)] = nsv
            return lossv_c

        lossv = lax.fori_loop(0, CHUNK // LANES, group_body, lossv)

    pltpu.sync_copy(ns_buf, nscore_out.at[pl.ds(base, ROWS)])
    loss_buf[...] = lossv
    pltpu.sync_copy(loss_buf, partial_out.at[wid])


_disc = functools.partial(
    pl.kernel,
    mesh=plsc.VectorSubcoreMesh(core_axis_name="c", subcore_axis_name="s"),
    compiler_params=pltpu.CompilerParams(needs_layout_passes=False),
    out_type=[
        jax.ShapeDtypeStruct((B,), jnp.float32),
        jax.ShapeDtypeStruct((NW, LANES), jnp.float32),
    ],
    scratch_types=(
        [pltpu.VMEM((CHUNK,), jnp.int32) for _ in range(12)]
        + [pltpu.VMEM((CHUNK, PR), jnp.float32) for _ in range(12)]
        + [pltpu.VMEM((ROWS,), jnp.float32),
           pltpu.VMEM((ROWS,), jnp.float32),
           pltpu.VMEM((LANES,), jnp.float32),
           pltpu.SemaphoreType.DMA,
           pltpu.SemaphoreType.DMA]
    ),
)(_disc_body)


def _repack_body(x_ref, o_ref, *, half):
    xt = x_ref[...].T
    o_ref[:, 0:64] = xt[0:half]
    o_ref[:, 64:128] = xt[half:2 * half]


def _make_repack(n_rows, blk):
    half = blk // 2
    grid = -(-n_rows // blk)
    return pl.pallas_call(
        functools.partial(_repack_body, half=half),
        grid=(grid,),
        in_specs=[pl.BlockSpec((D, blk), lambda i: (0, i))],
        out_specs=pl.BlockSpec((half, PR), lambda i: (i, 0)),
        out_shape=jax.ShapeDtypeStruct((grid * half, PR), jnp.float32),
    )


_ENT_BLK = 32768
_REL_BLK = 1000
_repack_ent = _make_repack(1000000, _ENT_BLK)
_repack_rel = _make_repack(1000, _REL_BLK)


def _sum_body(p_ref, o_ref):
    o_ref[0, 0] = jnp.sum(p_ref[...])


_sum_partials = pl.pallas_call(
    _sum_body,
    out_shape=jax.ShapeDtypeStruct((1, 1), jnp.float32),
    out_specs=pl.BlockSpec(memory_space=pltpu.SMEM),
)


def kernel(pos_h, pos_r, pos_t, neg_h, neg_r, neg_t, take,
           ent_emb_w, rel_emb_w, ent_transfer_w, rel_transfer_w):
    def map_idx(s, blk):
        s = s.astype(jnp.int32)
        half = blk // 2
        u = s % blk
        return (s // blk) * half + u % half, (u >= half).astype(jnp.int32) * D

    ent_srcs = [pos_h, pos_t, neg_h, neg_t]
    rel_srcs = [pos_r, neg_r]
    mph, aph = map_idx(pos_h, _ENT_BLK)
    mpt, apt = map_idx(pos_t, _ENT_BLK)
    mnh, anh = map_idx(neg_h, _ENT_BLK)
    mnt, ant = map_idx(neg_t, _ENT_BLK)
    mpr, apr = map_idx(pos_r, _REL_BLK)
    mnr, anr = map_idx(neg_r, _REL_BLK)
    phys = [mph, mpt, mpr, mnh, mnt, mnr]
    pars = [aph, apt, apr, anh, ant, anr]
    takef = take.astype(jnp.float32)
    ent2 = _repack_ent(ent_emb_w.T)
    etr2 = _repack_ent(ent_transfer_w.T)
    rel2 = _repack_rel(rel_emb_w.T)
    rtr2 = _repack_rel(rel_transfer_w.T)
    nscore, partials = _disc(*phys, *pars, takef,
                             ent2, rel2, etr2, rtr2)
    loss = _sum_partials(partials)[0, 0]
    return (loss, nscore)
